# baseline (speedup 1.0000x reference)
import functools

import jax
import jax.numpy as jnp
from jax import lax
from jax.experimental import pallas as pl
from jax.experimental.pallas import tpu as pltpu
from jax.experimental.pallas import tpu_sc as plsc

B, M, C = 4, 25000, 64
Z, X = 512, 512
XP = X + 1
NC, NS = 2, 16
NW = NC * NS
RPW = Z // NW
L = 16

CH = 2000
NCH = 12
TAIL = 1000
PLIST_CAP = M + 40
GCAP = 64
PEND_CAP = GCAP + 32
FLUSH_AT = 49


def _body(z_hbm, x_hbm, f_hbm, out_hbm,
          zbuf, xbuf, plist, pend, mbuf, fbuf, slab, touched,
          gsem, csem, osem):
    wid = lax.axis_index("s") * NC + lax.axis_index("c")
    z0 = wid * RPW
    iota = lax.iota(jnp.int32, L)
    zero_f = jnp.zeros((L,), jnp.float32)
    one_i = jnp.ones((L,), jnp.int32)

    _gdn = lax.GatherDimensionNumbers(
        offset_dims=(), collapsed_slice_dims=(0,), start_index_map=(0,))

    def vperm(v, idx):
        return lax.gather(v, idx[:, None], _gdn, slice_sizes=(1,),
                          mode=lax.GatherScatterMode.PROMISE_IN_BOUNDS)

    def vprefix(m):
        v = jnp.where(m, 1, 0)
        for s in (1, 2, 4, 8):
            sh = vperm(v, jnp.maximum(iota - s, 0))
            v = v + jnp.where(iota >= s, sh, 0)
        return v

    def scan_chunk(ncnt, base_m, nvalid, cb):
        def it(i, ncnt):
            zv = zbuf[cb, pl.ds(i * L, L)]
            xv = xbuf[cb, pl.ds(i * L, L)]
            lanem = (i * L + iota) < nvalid
            zrel = zv - z0
            inr = (zrel >= 0) & (zrel < RPW) & lanem

            def hit():
                psum = vprefix(inr)
                gm = base_m + i * L + iota
                p = gm | (xv << 17) | (zrel << 26)
                dest = jnp.where(inr, ncnt + psum - 1, PLIST_CAP - L + iota)
                plsc.store_scatter(plist, [dest], p)
                return ncnt + psum[L - 1]

            return lax.cond(jnp.any(inr), hit, lambda: ncnt)

        return lax.fori_loop(0, (nvalid + L - 1) // L, it, ncnt)

    def flush(cnt, sl):
        for k in range(GCAP // L):
            pk = pend[pl.ds(k * L, L)]
            valid = (k * L + iota) < cnt
            mbuf[pl.ds(k * L, L)] = jnp.where(valid, pk & 0x1FFFF, 0)
        pltpu.async_copy(f_hbm.at[mbuf], fbuf, gsem).wait()

        def pj_loop(j, _):
            pj = pend[pl.ds(j, L)][0]
            xj = lax.shift_right_logical(pj, 17) & 0x1FF
            xs = jnp.full((L,), xj, jnp.int32)
            tv = plsc.load_gather(touched.at[sl], [xs])
            first = tv == 0
            for q in range(C // L):
                cvec = q * L + iota
                fv = fbuf[j, pl.ds(q * L, L)]
                cur = plsc.load_gather(slab.at[sl], [cvec, xs])
                new = jnp.where(first, fv, jnp.maximum(cur, fv))
                plsc.store_scatter(slab.at[sl], [cvec, xs], new)
            tdest = jnp.where(iota == 0, xs, X + iota)
            plsc.store_scatter(touched.at[sl], [tdest], one_i)
            return 0

        lax.fori_loop(0, cnt, pj_loop, 0)

    def start_chunk_copy(b, g, cb):
        off = b * M + g * CH
        pltpu.async_copy(z_hbm.at[pl.ds(off, CH)], zbuf.at[cb], csem.at[cb])
        pltpu.async_copy(x_hbm.at[pl.ds(off, CH)], xbuf.at[cb], csem.at[cb])

    def wait_chunk_copy(cb):
        pltpu.make_async_copy(z_hbm.at[pl.ds(0, CH)], zbuf.at[cb],
                              csem.at[cb]).wait()
        pltpu.make_async_copy(x_hbm.at[pl.ds(0, CH)], xbuf.at[cb],
                              csem.at[cb]).wait()

    def per_batch(b, _):
        start_chunk_copy(b, 0, 0)

        def g_loop(g, n):
            cb = g % 2

            @pl.when(g + 1 < NCH)
            def _():
                start_chunk_copy(b, g + 1, 1 - cb)

            wait_chunk_copy(cb)
            return scan_chunk(n, b * M + g * CH, CH, cb)

        n = lax.fori_loop(0, NCH, g_loop, 0)
        toff = b * M + NCH * CH
        pltpu.sync_copy(z_hbm.at[pl.ds(toff, TAIL)], zbuf.at[0, pl.ds(0, TAIL)])
        pltpu.sync_copy(x_hbm.at[pl.ds(toff, TAIL)], xbuf.at[0, pl.ds(0, TAIL)])
        n = scan_chunk(n, toff, TAIL, 0)

        nch = (n + L - 1) // L

        def row(r, _):
            sl = r % 2
            g = b * RPW + r

            @pl.when(g >= 2)
            def _():
                pltpu.make_async_copy(slab.at[sl, :, pl.ds(0, X)],
                                      out_hbm.at[0, :, 0, :], osem.at[sl]).wait()

            def zc(c, _):
                for k in range(X // L):
                    slab[sl, c, pl.ds(k * L, L)] = zero_f
                return 0

            lax.fori_loop(0, C, zc, 0)
            for k in range((X + L) // L):
                touched[sl, pl.ds(k * L, L)] = jnp.zeros((L,), jnp.int32)

            def it(i, pc):
                pv = plist[pl.ds(i * L, L)]
                lanem = (i * L + iota) < n
                zrel = lax.shift_right_logical(pv, 26)
                match = (zrel == r) & lanem

                def hit():
                    psum = vprefix(match)
                    dest = jnp.where(match, pc + psum - 1, PEND_CAP - L + iota)
                    plsc.store_scatter(pend, [dest], pv)
                    npc = pc + psum[L - 1]

                    @pl.when(npc >= FLUSH_AT)
                    def _():
                        flush(npc, sl)

                    return jnp.where(npc >= FLUSH_AT, 0, npc)

                return lax.cond(jnp.any(match), hit, lambda: pc)

            pc = lax.fori_loop(0, nch, it, 0)

            @pl.when(pc > 0)
            def _():
                flush(pc, sl)

            pltpu.async_copy(slab.at[sl, :, pl.ds(0, X)],
                             out_hbm.at[b, :, z0 + r, :], osem.at[sl])
            return 0

        lax.fori_loop(0, RPW, row, 0)
        return 0

    lax.fori_loop(0, B, per_batch, 0)

    for sl in range(2):
        pltpu.make_async_copy(slab.at[sl, :, pl.ds(0, X)],
                              out_hbm.at[0, :, 0, :], osem.at[sl]).wait()


_sc_call = functools.partial(
    pl.kernel,
    out_type=jax.ShapeDtypeStruct((B, C, Z, X), jnp.float32),
    mesh=plsc.VectorSubcoreMesh(core_axis_name="c", subcore_axis_name="s"),
    compiler_params=pltpu.CompilerParams(
        needs_layout_passes=False, use_tc_tiling_on_sc=False),
    scratch_types=[
        pltpu.VMEM((2, CH), jnp.int32),
        pltpu.VMEM((2, CH), jnp.int32),
        pltpu.VMEM((PLIST_CAP,), jnp.int32),
        pltpu.VMEM((PEND_CAP,), jnp.int32),
        pltpu.VMEM((GCAP,), jnp.int32),
        pltpu.VMEM((GCAP, C), jnp.float32),
        pltpu.VMEM((2, C, XP), jnp.float32),
        pltpu.VMEM((2, X + L), jnp.int32),
        pltpu.SemaphoreType.DMA,
        pltpu.SemaphoreType.DMA((2,)),
        pltpu.SemaphoreType.DMA((2,)),
    ],
)(_body)


def kernel(voxel_features, voxel_coords):
    z = voxel_coords[:, :, 0].reshape(-1)
    x = voxel_coords[:, :, 2].reshape(-1)
    f = voxel_features.reshape(B * M, C)
    return _sc_call(z, x, f)


# speedup vs baseline: 2.0952x; 2.0952x over previous
import functools

import jax
import jax.numpy as jnp
from jax import lax
from jax.experimental import pallas as pl
from jax.experimental.pallas import tpu as pltpu
from jax.experimental.pallas import tpu_sc as plsc

B, M, C = 4, 25000, 64
Z, X = 512, 512
XP = X + 1
NC, NS = 2, 16
NW = NC * NS
RPW = Z // NW
L = 16

CH = 2000
NCH = 12
TAIL = 1000
PLIST_CAP = M + 40
GCAP = 64
PEND_CAP = GCAP + 32
FLUSH_AT = 49


def _body(z_hbm, x_hbm, f_hbm, out_hbm,
          zbuf, xbuf, plist, pend, mbuf, fbuf, slab, touched,
          gsem, csem, osem):
    wid = lax.axis_index("s") * NC + lax.axis_index("c")
    z0 = wid * RPW
    iota = lax.iota(jnp.int32, L)
    zero_f = jnp.zeros((L,), jnp.float32)
    one_i = jnp.ones((L,), jnp.int32)

    _gdn = lax.GatherDimensionNumbers(
        offset_dims=(), collapsed_slice_dims=(0,), start_index_map=(0,))

    def vperm(v, idx):
        return lax.gather(v, idx[:, None], _gdn, slice_sizes=(1,),
                          mode=lax.GatherScatterMode.PROMISE_IN_BOUNDS)

    def vprefix(m):
        v = jnp.where(m, 1, 0)
        for s in (1, 2, 4, 8):
            sh = vperm(v, jnp.maximum(iota - s, 0))
            v = v + jnp.where(iota >= s, sh, 0)
        return v

    def scan_chunk(ncnt, base_m, nvalid, cb):
        def it(i, ncnt):
            zv = zbuf[cb, pl.ds(i * L, L)]
            xv = xbuf[cb, pl.ds(i * L, L)]
            lanem = (i * L + iota) < nvalid
            zrel = zv - z0
            inr = (zrel >= 0) & (zrel < RPW) & lanem

            def hit():
                psum = vprefix(inr)
                gm = base_m + i * L + iota
                p = gm | (xv << 17) | (zrel << 26)
                dest = jnp.where(inr, ncnt + psum - 1, PLIST_CAP - L + iota)
                plsc.store_scatter(plist, [dest], p)
                return ncnt + psum[L - 1]

            return lax.cond(jnp.any(inr), hit, lambda: ncnt)

        return lax.fori_loop(0, (nvalid + L - 1) // L, it, ncnt)

    def flush(cnt, sl):
        for k in range(GCAP):
            pk = pend[pl.ds(k, L)][0]
            mk = jnp.minimum(pk & 0x1FFFF, B * M - 1)
            pltpu.async_copy(f_hbm.at[pl.ds(mk, 1)], fbuf.at[pl.ds(k, 1)], gsem)
        pltpu.make_async_copy(f_hbm.at[pl.ds(0, GCAP)], fbuf, gsem).wait()

        def pj_loop(j, _):
            pj = pend[pl.ds(j, L)][0]
            xj = lax.shift_right_logical(pj, 17) & 0x1FF
            xs = jnp.full((L,), xj, jnp.int32)
            tv = plsc.load_gather(touched.at[sl], [xs])
            first = tv == 0
            for q in range(C // L):
                cvec = q * L + iota
                fv = fbuf[j, pl.ds(q * L, L)]
                cur = plsc.load_gather(slab.at[sl], [cvec, xs])
                new = jnp.where(first, fv, jnp.maximum(cur, fv))
                plsc.store_scatter(slab.at[sl], [cvec, xs], new)
            tdest = jnp.where(iota == 0, xs, X + iota)
            plsc.store_scatter(touched.at[sl], [tdest], one_i)
            return 0

        lax.fori_loop(0, cnt, pj_loop, 0)

    def start_chunk_copy(b, g, cb):
        off = b * M + g * CH
        pltpu.async_copy(z_hbm.at[pl.ds(off, CH)], zbuf.at[cb], csem.at[cb])
        pltpu.async_copy(x_hbm.at[pl.ds(off, CH)], xbuf.at[cb], csem.at[cb])

    def wait_chunk_copy(cb):
        pltpu.make_async_copy(z_hbm.at[pl.ds(0, CH)], zbuf.at[cb],
                              csem.at[cb]).wait()
        pltpu.make_async_copy(x_hbm.at[pl.ds(0, CH)], xbuf.at[cb],
                              csem.at[cb]).wait()

    def per_batch(b, _):
        start_chunk_copy(b, 0, 0)

        def g_loop(g, n):
            cb = g % 2

            @pl.when(g + 1 < NCH)
            def _():
                start_chunk_copy(b, g + 1, 1 - cb)

            wait_chunk_copy(cb)
            return scan_chunk(n, b * M + g * CH, CH, cb)

        n = lax.fori_loop(0, NCH, g_loop, 0)
        toff = b * M + NCH * CH
        pltpu.sync_copy(z_hbm.at[pl.ds(toff, TAIL)], zbuf.at[0, pl.ds(0, TAIL)])
        pltpu.sync_copy(x_hbm.at[pl.ds(toff, TAIL)], xbuf.at[0, pl.ds(0, TAIL)])
        n = scan_chunk(n, toff, TAIL, 0)

        nch = (n + L - 1) // L

        def row(r, _):
            sl = r % 2
            g = b * RPW + r

            @pl.when(g >= 2)
            def _():
                pltpu.make_async_copy(slab.at[sl, :, pl.ds(0, X)],
                                      out_hbm.at[0, :, 0, :], osem.at[sl]).wait()

            def zc(c, _):
                for k in range(X // L):
                    slab[sl, c, pl.ds(k * L, L)] = zero_f
                return 0

            lax.fori_loop(0, C, zc, 0)
            for k in range((X + L) // L):
                touched[sl, pl.ds(k * L, L)] = jnp.zeros((L,), jnp.int32)

            def it(i, pc):
                pv = plist[pl.ds(i * L, L)]
                lanem = (i * L + iota) < n
                zrel = lax.shift_right_logical(pv, 26)
                match = (zrel == r) & lanem

                def hit():
                    psum = vprefix(match)
                    dest = jnp.where(match, pc + psum - 1, PEND_CAP - L + iota)
                    plsc.store_scatter(pend, [dest], pv)
                    npc = pc + psum[L - 1]

                    @pl.when(npc >= FLUSH_AT)
                    def _():
                        flush(npc, sl)

                    return jnp.where(npc >= FLUSH_AT, 0, npc)

                return lax.cond(jnp.any(match), hit, lambda: pc)

            pc = lax.fori_loop(0, nch, it, 0)

            @pl.when(pc > 0)
            def _():
                flush(pc, sl)

            pltpu.async_copy(slab.at[sl, :, pl.ds(0, X)],
                             out_hbm.at[b, :, z0 + r, :], osem.at[sl])
            return 0

        lax.fori_loop(0, RPW, row, 0)
        return 0

    lax.fori_loop(0, B, per_batch, 0)

    for sl in range(2):
        pltpu.make_async_copy(slab.at[sl, :, pl.ds(0, X)],
                              out_hbm.at[0, :, 0, :], osem.at[sl]).wait()


_sc_call = functools.partial(
    pl.kernel,
    out_type=jax.ShapeDtypeStruct((B, C, Z, X), jnp.float32),
    mesh=plsc.VectorSubcoreMesh(core_axis_name="c", subcore_axis_name="s"),
    compiler_params=pltpu.CompilerParams(
        needs_layout_passes=False, use_tc_tiling_on_sc=False),
    scratch_types=[
        pltpu.VMEM((2, CH), jnp.int32),
        pltpu.VMEM((2, CH), jnp.int32),
        pltpu.VMEM((PLIST_CAP,), jnp.int32),
        pltpu.VMEM((PEND_CAP,), jnp.int32),
        pltpu.VMEM((GCAP,), jnp.int32),
        pltpu.VMEM((GCAP, C), jnp.float32),
        pltpu.VMEM((2, C, XP), jnp.float32),
        pltpu.VMEM((2, X + L), jnp.int32),
        pltpu.SemaphoreType.DMA,
        pltpu.SemaphoreType.DMA((2,)),
        pltpu.SemaphoreType.DMA((2,)),
    ],
)(_body)


def kernel(voxel_features, voxel_coords):
    z = voxel_coords[:, :, 0].reshape(-1)
    x = voxel_coords[:, :, 2].reshape(-1)
    f = voxel_features.reshape(B * M, C)
    return _sc_call(z, x, f)


# speedup vs baseline: 2.5261x; 1.2057x over previous
import functools

import jax
import jax.numpy as jnp
from jax import lax
from jax.experimental import pallas as pl
from jax.experimental.pallas import tpu as pltpu
from jax.experimental.pallas import tpu_sc as plsc

B, M, C = 4, 25000, 64
Z, X = 512, 512
XP = X + 1
NC, NS = 2, 16
NW = NC * NS
WPB = NW // B
RPW = Z // WPB
L = 16

CH = 2000
NCH = 12
TAIL = 1000
SCAP = M + 24
GCAP = 64
TMPC = 32


def _body(z_hbm, x_hbm, f_hbm, out_hbm,
          zbuf, xbuf, srt, tmp, counts, starts, offs, fbuf, slab, touched,
          gsem, csem, osem):
    wid = lax.axis_index("s") * NC + lax.axis_index("c")
    b = wid // WPB
    z0 = (wid % WPB) * RPW
    iota = lax.iota(jnp.int32, L)
    zero_f = jnp.zeros((L,), jnp.float32)
    zero_i = jnp.zeros((L,), jnp.int32)
    one0 = jnp.where(iota == 0, 1, 0)

    _gdn = lax.GatherDimensionNumbers(
        offset_dims=(), collapsed_slice_dims=(0,), start_index_map=(0,))

    def vperm(v, idx):
        return lax.gather(v, idx[:, None], _gdn, slice_sizes=(1,),
                          mode=lax.GatherScatterMode.PROMISE_IN_BOUNDS)

    def vprefixi(v):
        for s in (1, 2, 4, 8):
            sh = vperm(v, jnp.maximum(iota - s, 0))
            v = v + jnp.where(iota >= s, sh, 0)
        return v

    def compact(vals, mask):
        psum = vprefixi(jnp.where(mask, 1, 0))
        dest = jnp.where(mask, psum - 1, L + iota)
        plsc.store_scatter(tmp, [dest], vals)
        return psum[L - 1]

    def start_z_copy(g, cb):
        pltpu.async_copy(z_hbm.at[pl.ds(b * M + g * CH, CH)], zbuf.at[cb],
                         csem.at[cb])

    def start_x_copy(g, cb):
        pltpu.async_copy(x_hbm.at[pl.ds(b * M + g * CH, CH)], xbuf.at[cb],
                         csem.at[cb])

    def wait_copies(cb, refs):
        for ref in refs:
            pltpu.make_async_copy(z_hbm.at[pl.ds(0, CH)], ref.at[cb],
                                  csem.at[cb]).wait()

    for k in range(80 // L):
        counts[pl.ds(k * L, L)] = zero_i

    start_z_copy(0, 0)

    def scanA_chunk(g, nvalid, cb):
        def it(i, _):
            zv = zbuf[cb, pl.ds(i * L, L)]
            lanem = (i * L + iota) < nvalid
            zrel = zv - z0
            inr = (zrel >= 0) & (zrel < RPW) & lanem

            def hit():
                cnt = compact(zrel, inr)

                def cj(j, _):
                    zr = tmp[pl.ds(j, L)][0]
                    w = counts[pl.ds(zr, L)]
                    counts[pl.ds(zr, L)] = w + one0
                    return 0

                lax.fori_loop(0, cnt, cj, 0)
                return 0

            lax.cond(jnp.any(inr), hit, lambda: 0)
            return 0

        lax.fori_loop(0, (nvalid + L - 1) // L, it, 0)

    def gA(g, _):
        cb = g % 2

        @pl.when(g + 1 < NCH)
        def _():
            start_z_copy(g + 1, 1 - cb)

        wait_copies(cb, [zbuf])
        scanA_chunk(g, CH, cb)
        return 0

    lax.fori_loop(0, NCH, gA, 0)
    pltpu.sync_copy(z_hbm.at[pl.ds(b * M + NCH * CH, TAIL)],
                    zbuf.at[0, pl.ds(0, TAIL)])
    scanA_chunk(NCH, TAIL, 0)

    base = 0
    for gix in range(RPW // L):
        cg = counts[pl.ds(gix * L, L)]
        incl = vprefixi(cg)
        excl = incl - cg + base
        starts[pl.ds(gix * L, L)] = excl
        offs[pl.ds(gix * L, L)] = excl
        base = base + incl[L - 1]
    starts[pl.ds(RPW, L)] = jnp.full((L,), base, jnp.int32)

    start_z_copy(0, 0)
    start_x_copy(0, 0)

    def scanB_chunk(g, nvalid, cb):
        def it(i, _):
            zv = zbuf[cb, pl.ds(i * L, L)]
            xv = xbuf[cb, pl.ds(i * L, L)]
            lanem = (i * L + iota) < nvalid
            zrel = zv - z0
            inr = (zrel >= 0) & (zrel < RPW) & lanem

            def hit():
                m = g * CH + i * L + iota
                p = m | (xv << 15) | (zrel << 24)
                cnt = compact(p, inr)

                def cj(j, _):
                    pk = tmp[pl.ds(j, L)][0]
                    zr = lax.shift_right_logical(pk, 24)
                    w = offs[pl.ds(zr, L)]
                    off = w[0]
                    dest = jnp.where(iota == 0, off, SCAP - L + iota)
                    plsc.store_scatter(srt, [dest],
                                       jnp.full((L,), pk, jnp.int32))
                    offs[pl.ds(zr, L)] = w + one0
                    return 0

                lax.fori_loop(0, cnt, cj, 0)
                return 0

            lax.cond(jnp.any(inr), hit, lambda: 0)
            return 0

        lax.fori_loop(0, (nvalid + L - 1) // L, it, 0)

    def gB(g, _):
        cb = g % 2

        @pl.when(g + 1 < NCH)
        def _():
            start_z_copy(g + 1, 1 - cb)
            start_x_copy(g + 1, 1 - cb)

        wait_copies(cb, [zbuf, xbuf])
        scanB_chunk(g, CH, cb)
        return 0

    lax.fori_loop(0, NCH, gB, 0)
    pltpu.sync_copy(z_hbm.at[pl.ds(b * M + NCH * CH, TAIL)],
                    zbuf.at[0, pl.ds(0, TAIL)])
    pltpu.sync_copy(x_hbm.at[pl.ds(b * M + NCH * CH, TAIL)],
                    xbuf.at[0, pl.ds(0, TAIL)])
    scanB_chunk(NCH, TAIL, 0)

    def row(r, _):
        sl = r % 2

        @pl.when(r >= 2)
        def _():
            pltpu.make_async_copy(slab.at[sl, :, pl.ds(0, X)],
                                  out_hbm.at[0, :, 0, :], osem.at[sl]).wait()

        def zc(c, _):
            for k in range(X // L):
                slab[sl, c, pl.ds(k * L, L)] = zero_f
            return 0

        lax.fori_loop(0, C, zc, 0)
        for k in range((X + L) // L):
            touched[sl, pl.ds(k * L, L)] = zero_i

        sr = starts[pl.ds(r, L)][0]
        er = starts[pl.ds(r + 1, L)][0]
        ng = (er - sr + GCAP - 1) // GCAP

        def grp(k, _):
            gstart = sr + k * GCAP
            cnt = jnp.minimum(er - gstart, GCAP)
            for kk in range(GCAP):
                idx = jnp.minimum(gstart + kk, er - 1)
                pk = srt[pl.ds(idx, L)][0]
                mg = b * M + (pk & 0x7FFF)
                pltpu.async_copy(f_hbm.at[pl.ds(mg, 1)],
                                 fbuf.at[pl.ds(kk, 1)], gsem)
            pltpu.make_async_copy(f_hbm.at[pl.ds(0, GCAP)], fbuf, gsem).wait()

            def pj(j, _):
                pj_ = srt[pl.ds(gstart + j, L)][0]
                xj = lax.shift_right_logical(pj_, 15) & 0x1FF
                xs = jnp.full((L,), xj, jnp.int32)
                tv = plsc.load_gather(touched.at[sl], [xs])
                first = tv == 0
                for q in range(C // L):
                    cvec = q * L + iota
                    fv = fbuf[j, pl.ds(q * L, L)]
                    cur = plsc.load_gather(slab.at[sl], [cvec, xs])
                    new = jnp.where(first, fv, jnp.maximum(cur, fv))
                    plsc.store_scatter(slab.at[sl], [cvec, xs], new)
                tdest = jnp.where(iota == 0, xs, X + iota)
                plsc.store_scatter(touched.at[sl], [tdest],
                                   jnp.ones((L,), jnp.int32))
                return 0

            lax.fori_loop(0, cnt, pj, 0)
            return 0

        lax.fori_loop(0, ng, grp, 0)

        pltpu.async_copy(slab.at[sl, :, pl.ds(0, X)],
                         out_hbm.at[b, :, z0 + r, :], osem.at[sl])
        return 0

    lax.fori_loop(0, RPW, row, 0)

    for sl in range(2):
        pltpu.make_async_copy(slab.at[sl, :, pl.ds(0, X)],
                              out_hbm.at[0, :, 0, :], osem.at[sl]).wait()


_sc_call = functools.partial(
    pl.kernel,
    out_type=jax.ShapeDtypeStruct((B, C, Z, X), jnp.float32),
    mesh=plsc.VectorSubcoreMesh(core_axis_name="c", subcore_axis_name="s"),
    compiler_params=pltpu.CompilerParams(
        needs_layout_passes=False, use_tc_tiling_on_sc=False),
    scratch_types=[
        pltpu.VMEM((2, CH), jnp.int32),
        pltpu.VMEM((2, CH), jnp.int32),
        pltpu.VMEM((SCAP,), jnp.int32),
        pltpu.VMEM((TMPC,), jnp.int32),
        pltpu.VMEM((80,), jnp.int32),
        pltpu.VMEM((80,), jnp.int32),
        pltpu.VMEM((80,), jnp.int32),
        pltpu.VMEM((GCAP, C), jnp.float32),
        pltpu.VMEM((2, C, XP), jnp.float32),
        pltpu.VMEM((2, X + L), jnp.int32),
        pltpu.SemaphoreType.DMA,
        pltpu.SemaphoreType.DMA((2,)),
        pltpu.SemaphoreType.DMA((2,)),
    ],
)(_body)


def kernel(voxel_features, voxel_coords):
    z = voxel_coords[:, :, 0].reshape(-1)
    x = voxel_coords[:, :, 2].reshape(-1)
    f = voxel_features.reshape(B * M, C)
    return _sc_call(z, x, f)


# speedup vs baseline: 2.7764x; 1.0991x over previous
import functools

import jax
import jax.numpy as jnp
from jax import lax
from jax.experimental import pallas as pl
from jax.experimental.pallas import tpu as pltpu
from jax.experimental.pallas import tpu_sc as plsc

B, M, C = 4, 25000, 64
Z, X = 512, 512
XP = X + 1
NC, NS = 2, 16
NW = NC * NS
WPB = NW // B
RPW = Z // WPB
L = 16

CH = 2000
NCH = 12
TAIL = 1000
SCAP = M + 88
GCAP = 64
TMPC = 32


def _body(z_hbm, x_hbm, f_hbm, out_hbm,
          zbuf, xbuf, srt, tmp, counts, starts, offs, fbuf, slab, touched,
          gsem, csem, osem):
    wid = lax.axis_index("s") * NC + lax.axis_index("c")
    b = wid // WPB
    z0 = (wid % WPB) * RPW
    iota = lax.iota(jnp.int32, L)
    zero_f = jnp.zeros((L,), jnp.float32)
    zero_i = jnp.zeros((L,), jnp.int32)
    one0 = jnp.where(iota == 0, 1, 0)

    _gdn = lax.GatherDimensionNumbers(
        offset_dims=(), collapsed_slice_dims=(0,), start_index_map=(0,))

    def vperm(v, idx):
        return lax.gather(v, idx[:, None], _gdn, slice_sizes=(1,),
                          mode=lax.GatherScatterMode.PROMISE_IN_BOUNDS)

    def vprefixi(v):
        for s in (1, 2, 4, 8):
            sh = vperm(v, jnp.maximum(iota - s, 0))
            v = v + jnp.where(iota >= s, sh, 0)
        return v

    def compact(vals, mask):
        psum = vprefixi(jnp.where(mask, 1, 0))
        dest = jnp.where(mask, psum - 1, L + iota)
        plsc.store_scatter(tmp, [dest], vals)
        return psum[L - 1]

    def start_z_copy(g, cb):
        pltpu.async_copy(z_hbm.at[pl.ds(b * M + g * CH, CH)], zbuf.at[cb],
                         csem.at[cb])

    def start_x_copy(g, cb):
        pltpu.async_copy(x_hbm.at[pl.ds(b * M + g * CH, CH)], xbuf.at[cb],
                         csem.at[cb])

    def wait_copies(cb, refs):
        for ref in refs:
            pltpu.make_async_copy(z_hbm.at[pl.ds(0, CH)], ref.at[cb],
                                  csem.at[cb]).wait()

    for k in range(80 // L):
        counts[pl.ds(k * L, L)] = zero_i

    start_z_copy(0, 0)

    def scanA_chunk(g, nvalid, cb):
        def it(i, _):
            zv = zbuf[cb, pl.ds(i * L, L)]
            lanem = (i * L + iota) < nvalid
            zrel = zv - z0
            inr = (zrel >= 0) & (zrel < RPW) & lanem

            def hit():
                cnt = compact(zrel, inr)

                def cj(j, _):
                    zr = tmp[pl.ds(j, L)][0]
                    w = counts[pl.ds(zr, L)]
                    counts[pl.ds(zr, L)] = w + one0
                    return 0

                lax.fori_loop(0, cnt, cj, 0)
                return 0

            lax.cond(jnp.any(inr), hit, lambda: 0)
            return 0

        lax.fori_loop(0, (nvalid + L - 1) // L, it, 0)

    def gA(g, _):
        cb = g % 2

        @pl.when(g + 1 < NCH)
        def _():
            start_z_copy(g + 1, 1 - cb)

        wait_copies(cb, [zbuf])
        scanA_chunk(g, CH, cb)
        return 0

    lax.fori_loop(0, NCH, gA, 0)
    pltpu.sync_copy(z_hbm.at[pl.ds(b * M + NCH * CH, TAIL)],
                    zbuf.at[0, pl.ds(0, TAIL)])
    scanA_chunk(NCH, TAIL, 0)

    base = 0
    for gix in range(RPW // L):
        cg = counts[pl.ds(gix * L, L)]
        incl = vprefixi(cg)
        excl = incl - cg + base
        starts[pl.ds(gix * L, L)] = excl
        offs[pl.ds(gix * L, L)] = excl
        base = base + incl[L - 1]
    starts[pl.ds(RPW, L)] = jnp.full((L,), base, jnp.int32)

    start_z_copy(0, 0)
    start_x_copy(0, 0)

    def scanB_chunk(g, nvalid, cb):
        def it(i, _):
            zv = zbuf[cb, pl.ds(i * L, L)]
            xv = xbuf[cb, pl.ds(i * L, L)]
            lanem = (i * L + iota) < nvalid
            zrel = zv - z0
            inr = (zrel >= 0) & (zrel < RPW) & lanem

            def hit():
                m = g * CH + i * L + iota
                p = m | (xv << 15) | (zrel << 24)
                cnt = compact(p, inr)

                def cj(j, _):
                    pk = tmp[pl.ds(j, L)][0]
                    zr = lax.shift_right_logical(pk, 24)
                    w = offs[pl.ds(zr, L)]
                    off = w[0]
                    dest = jnp.where(iota == 0, off, SCAP - L + iota)
                    plsc.store_scatter(srt, [dest],
                                       jnp.full((L,), pk, jnp.int32))
                    offs[pl.ds(zr, L)] = w + one0
                    return 0

                lax.fori_loop(0, cnt, cj, 0)
                return 0

            lax.cond(jnp.any(inr), hit, lambda: 0)
            return 0

        lax.fori_loop(0, (nvalid + L - 1) // L, it, 0)

    def gB(g, _):
        cb = g % 2

        @pl.when(g + 1 < NCH)
        def _():
            start_z_copy(g + 1, 1 - cb)
            start_x_copy(g + 1, 1 - cb)

        wait_copies(cb, [zbuf, xbuf])
        scanB_chunk(g, CH, cb)
        return 0

    lax.fori_loop(0, NCH, gB, 0)
    pltpu.sync_copy(z_hbm.at[pl.ds(b * M + NCH * CH, TAIL)],
                    zbuf.at[0, pl.ds(0, TAIL)])
    pltpu.sync_copy(x_hbm.at[pl.ds(b * M + NCH * CH, TAIL)],
                    xbuf.at[0, pl.ds(0, TAIL)])
    scanB_chunk(NCH, TAIL, 0)

    def row(r, _):
        sl = r % 2

        @pl.when(r >= 2)
        def _():
            pltpu.make_async_copy(slab.at[sl, :, pl.ds(0, X)],
                                  out_hbm.at[0, :, 0, :], osem.at[sl]).wait()

        sr = starts[pl.ds(r, L)][0]
        er = starts[pl.ds(r + 1, L)][0]
        ng = (er - sr + GCAP - 1) // GCAP

        def fire(gstart, par):
            for w in range(GCAP // L):
                win = srt[pl.ds(gstart + w * L, L)]
                for t in range(L):
                    mg = jnp.minimum(b * M + (win[t] & 0x7FFF), B * M - 1)
                    pltpu.async_copy(f_hbm.at[pl.ds(mg, 1)],
                                     fbuf.at[par, pl.ds(w * L + t, 1)],
                                     gsem.at[par])

        @pl.when(ng > 0)
        def _():
            fire(sr, 0)

        def zc(c, _):
            for k in range(X // L):
                slab[sl, c, pl.ds(k * L, L)] = zero_f
            return 0

        lax.fori_loop(0, C, zc, 0)
        for k in range((X + L) // L):
            touched[sl, pl.ds(k * L, L)] = zero_i

        def grp(k, _):
            par = k % 2
            pltpu.make_async_copy(f_hbm.at[pl.ds(0, GCAP)], fbuf.at[par],
                                  gsem.at[par]).wait()

            @pl.when(k + 1 < ng)
            def _():
                fire(sr + (k + 1) * GCAP, (k + 1) % 2)

            gstart = sr + k * GCAP
            cnt = jnp.minimum(er - gstart, GCAP)

            def pj(j, _):
                pj_ = srt[pl.ds(gstart + j, L)][0]
                xj = lax.shift_right_logical(pj_, 15) & 0x1FF
                xs = jnp.full((L,), xj, jnp.int32)
                tv = plsc.load_gather(touched.at[sl], [xs])
                first = tv == 0
                for q in range(C // L):
                    cvec = q * L + iota
                    fv = fbuf[par, j, pl.ds(q * L, L)]
                    cur = plsc.load_gather(slab.at[sl], [cvec, xs])
                    new = jnp.where(first, fv, jnp.maximum(cur, fv))
                    plsc.store_scatter(slab.at[sl], [cvec, xs], new)
                tdest = jnp.where(iota == 0, xs, X + iota)
                plsc.store_scatter(touched.at[sl], [tdest],
                                   jnp.ones((L,), jnp.int32))
                return 0

            lax.fori_loop(0, cnt, pj, 0)
            return 0

        lax.fori_loop(0, ng, grp, 0)

        pltpu.async_copy(slab.at[sl, :, pl.ds(0, X)],
                         out_hbm.at[b, :, z0 + r, :], osem.at[sl])
        return 0

    lax.fori_loop(0, RPW, row, 0)

    for sl in range(2):
        pltpu.make_async_copy(slab.at[sl, :, pl.ds(0, X)],
                              out_hbm.at[0, :, 0, :], osem.at[sl]).wait()


_sc_call = functools.partial(
    pl.kernel,
    out_type=jax.ShapeDtypeStruct((B, C, Z, X), jnp.float32),
    mesh=plsc.VectorSubcoreMesh(core_axis_name="c", subcore_axis_name="s"),
    compiler_params=pltpu.CompilerParams(
        needs_layout_passes=False, use_tc_tiling_on_sc=False),
    scratch_types=[
        pltpu.VMEM((2, CH), jnp.int32),
        pltpu.VMEM((2, CH), jnp.int32),
        pltpu.VMEM((SCAP,), jnp.int32),
        pltpu.VMEM((TMPC,), jnp.int32),
        pltpu.VMEM((80,), jnp.int32),
        pltpu.VMEM((80,), jnp.int32),
        pltpu.VMEM((80,), jnp.int32),
        pltpu.VMEM((2, GCAP, C), jnp.float32),
        pltpu.VMEM((2, C, XP), jnp.float32),
        pltpu.VMEM((2, X + L), jnp.int32),
        pltpu.SemaphoreType.DMA((2,)),
        pltpu.SemaphoreType.DMA((2,)),
        pltpu.SemaphoreType.DMA((2,)),
    ],
)(_body)


def kernel(voxel_features, voxel_coords):
    z = voxel_coords[:, :, 0].reshape(-1)
    x = voxel_coords[:, :, 2].reshape(-1)
    f = voxel_features.reshape(B * M, C)
    return _sc_call(z, x, f)


# speedup vs baseline: 2.8104x; 1.0122x over previous
import functools

import jax
import jax.numpy as jnp
from jax import lax
from jax.experimental import pallas as pl
from jax.experimental.pallas import tpu as pltpu
from jax.experimental.pallas import tpu_sc as plsc

B, M, C = 4, 25000, 64
Z, X = 512, 512
XP = X + 1
NC, NS = 2, 16
NW = NC * NS
WPB = NW // B
RPW = Z // WPB
L = 16

CH = 2000
NCH = 12
TAIL = 1000
SCAP = M + 88
GCAP = 64
TMPC = 32


def _body(z_hbm, x_hbm, f_hbm, out_hbm,
          zbuf, xbuf, srt, tmp, counts, starts, offs, fbuf, slab, touched,
          gsem, csem, osem):
    wid = lax.axis_index("s") * NC + lax.axis_index("c")
    b = wid // WPB
    z0 = (wid % WPB) * RPW
    iota = lax.iota(jnp.int32, L)
    zero_f = jnp.zeros((L,), jnp.float32)
    zero_i = jnp.zeros((L,), jnp.int32)
    one0 = jnp.where(iota == 0, 1, 0)

    _gdn = lax.GatherDimensionNumbers(
        offset_dims=(), collapsed_slice_dims=(0,), start_index_map=(0,))

    def vperm(v, idx):
        return lax.gather(v, idx[:, None], _gdn, slice_sizes=(1,),
                          mode=lax.GatherScatterMode.PROMISE_IN_BOUNDS)

    def vprefixi(v):
        for s in (1, 2, 4, 8):
            sh = vperm(v, jnp.maximum(iota - s, 0))
            v = v + jnp.where(iota >= s, sh, 0)
        return v

    def compact(vals, mask):
        psum = vprefixi(jnp.where(mask, 1, 0))
        dest = jnp.where(mask, psum - 1, L + iota)
        plsc.store_scatter(tmp, [dest], vals)
        return psum[L - 1]

    def start_z_copy(g, cb):
        pltpu.async_copy(z_hbm.at[pl.ds(b * M + g * CH, CH)], zbuf.at[cb],
                         csem.at[cb])

    def start_x_copy(g, cb):
        pltpu.async_copy(x_hbm.at[pl.ds(b * M + g * CH, CH)], xbuf.at[cb],
                         csem.at[cb])

    def wait_copies(cb, refs):
        for ref in refs:
            pltpu.make_async_copy(z_hbm.at[pl.ds(0, CH)], ref.at[cb],
                                  csem.at[cb]).wait()

    for k in range(80 // L):
        counts[pl.ds(k * L, L)] = zero_i

    start_z_copy(0, 0)

    def scanA_chunk(g, nvalid, cb):
        def it(i, _):
            zv = zbuf[cb, pl.ds(i * L, L)]
            lanem = (i * L + iota) < nvalid
            zrel = zv - z0
            inr = (zrel >= 0) & (zrel < RPW) & lanem

            def hit():
                cnt = compact(zrel, inr)

                def cj(j, _):
                    zr = tmp[pl.ds(j, L)][0]
                    w = counts[pl.ds(zr, L)]
                    counts[pl.ds(zr, L)] = w + one0
                    return 0

                lax.fori_loop(0, cnt, cj, 0)
                return 0

            lax.cond(jnp.any(inr), hit, lambda: 0)
            return 0

        lax.fori_loop(0, (nvalid + L - 1) // L, it, 0)

    def gA(g, _):
        cb = g % 2

        @pl.when(g + 1 < NCH)
        def _():
            start_z_copy(g + 1, 1 - cb)

        wait_copies(cb, [zbuf])
        scanA_chunk(g, CH, cb)
        return 0

    lax.fori_loop(0, NCH, gA, 0)
    pltpu.sync_copy(z_hbm.at[pl.ds(b * M + NCH * CH, TAIL)],
                    zbuf.at[0, pl.ds(0, TAIL)])
    scanA_chunk(NCH, TAIL, 0)

    base = 0
    for gix in range(RPW // L):
        cg = counts[pl.ds(gix * L, L)]
        incl = vprefixi(cg)
        excl = incl - cg + base
        starts[pl.ds(gix * L, L)] = excl
        offs[pl.ds(gix * L, L)] = excl
        base = base + incl[L - 1]
    starts[pl.ds(RPW, L)] = jnp.full((L,), base, jnp.int32)
    starts[pl.ds(RPW + L, L)] = jnp.full((L,), base, jnp.int32)

    start_z_copy(0, 0)
    start_x_copy(0, 0)

    def scanB_chunk(g, nvalid, cb):
        def it(i, _):
            zv = zbuf[cb, pl.ds(i * L, L)]
            xv = xbuf[cb, pl.ds(i * L, L)]
            lanem = (i * L + iota) < nvalid
            zrel = zv - z0
            inr = (zrel >= 0) & (zrel < RPW) & lanem

            def hit():
                m = g * CH + i * L + iota
                p = m | (xv << 15) | (zrel << 24)
                cnt = compact(p, inr)

                def cj(j, _):
                    pk = tmp[pl.ds(j, L)][0]
                    zr = lax.shift_right_logical(pk, 24)
                    w = offs[pl.ds(zr, L)]
                    off = w[0]
                    dest = jnp.where(iota == 0, off, SCAP - L + iota)
                    plsc.store_scatter(srt, [dest],
                                       jnp.full((L,), pk, jnp.int32))
                    offs[pl.ds(zr, L)] = w + one0
                    return 0

                lax.fori_loop(0, cnt, cj, 0)
                return 0

            lax.cond(jnp.any(inr), hit, lambda: 0)
            return 0

        lax.fori_loop(0, (nvalid + L - 1) // L, it, 0)

    def gB(g, _):
        cb = g % 2

        @pl.when(g + 1 < NCH)
        def _():
            start_z_copy(g + 1, 1 - cb)
            start_x_copy(g + 1, 1 - cb)

        wait_copies(cb, [zbuf, xbuf])
        scanB_chunk(g, CH, cb)
        return 0

    lax.fori_loop(0, NCH, gB, 0)
    pltpu.sync_copy(z_hbm.at[pl.ds(b * M + NCH * CH, TAIL)],
                    zbuf.at[0, pl.ds(0, TAIL)])
    pltpu.sync_copy(x_hbm.at[pl.ds(b * M + NCH * CH, TAIL)],
                    xbuf.at[0, pl.ds(0, TAIL)])
    scanB_chunk(NCH, TAIL, 0)

    def row(r, bp):
        sl = r % 2

        @pl.when(r >= 2)
        def _():
            pltpu.make_async_copy(slab.at[sl, :, pl.ds(0, X)],
                                  out_hbm.at[0, :, 0, :], osem.at[sl]).wait()

        sr = starts[pl.ds(r, L)][0]
        er = starts[pl.ds(r + 1, L)][0]
        ng = (er - sr + GCAP - 1) // GCAP

        def fire(gstart, par):
            for w in range(GCAP // L):
                win = srt[pl.ds(gstart + w * L, L)]
                for t in range(L):
                    mg = jnp.minimum(b * M + (win[t] & 0x7FFF), B * M - 1)
                    pltpu.async_copy(f_hbm.at[pl.ds(mg, 1)],
                                     fbuf.at[par, pl.ds(w * L + t, 1)],
                                     gsem.at[par])


        def zc(c, _):
            for k in range(X // L):
                slab[sl, c, pl.ds(k * L, L)] = zero_f
            return 0

        lax.fori_loop(0, C, zc, 0)
        for k in range((X + L) // L):
            touched[sl, pl.ds(k * L, L)] = zero_i

        def next_row_fire():
            nsr = starts[pl.ds(r + 1, L)][0]
            ner = starts[pl.ds(r + 2, L)][0]

            @pl.when((r + 1 < RPW) & (ner > nsr))
            def _():
                fire(nsr, (bp + ng) % 2)

        def grp(k, _):
            par = (bp + k) % 2
            pltpu.make_async_copy(f_hbm.at[pl.ds(0, GCAP)], fbuf.at[par],
                                  gsem.at[par]).wait()

            @pl.when(k + 1 < ng)
            def _():
                fire(sr + (k + 1) * GCAP, (bp + k + 1) % 2)

            @pl.when(k + 1 == ng)
            def _():
                next_row_fire()

            gstart = sr + k * GCAP
            cnt = jnp.minimum(er - gstart, GCAP)

            def pj(j, _):
                pj_ = srt[pl.ds(gstart + j, L)][0]
                xj = lax.shift_right_logical(pj_, 15) & 0x1FF
                xs = jnp.full((L,), xj, jnp.int32)
                tv = plsc.load_gather(touched.at[sl], [xs])
                first = tv == 0
                for q in range(C // L):
                    cvec = q * L + iota
                    fv = fbuf[par, j, pl.ds(q * L, L)]
                    cur = plsc.load_gather(slab.at[sl], [cvec, xs])
                    new = jnp.where(first, fv, jnp.maximum(cur, fv))
                    plsc.store_scatter(slab.at[sl], [cvec, xs], new)
                tdest = jnp.where(iota == 0, xs, X + iota)
                plsc.store_scatter(touched.at[sl], [tdest],
                                   jnp.ones((L,), jnp.int32))
                return 0

            lax.fori_loop(0, cnt, pj, 0)
            return 0

        lax.fori_loop(0, ng, grp, 0)

        @pl.when(ng == 0)
        def _():
            next_row_fire()

        pltpu.async_copy(slab.at[sl, :, pl.ds(0, X)],
                         out_hbm.at[b, :, z0 + r, :], osem.at[sl])
        return (bp + ng) % 2

    sr0 = starts[pl.ds(0, L)][0]
    er0 = starts[pl.ds(1, L)][0]

    @pl.when(er0 > sr0)
    def _():
        def fire0(w):
            win = srt[pl.ds(sr0 + w * L, L)]
            for t in range(L):
                mg = jnp.minimum(b * M + (win[t] & 0x7FFF), B * M - 1)
                pltpu.async_copy(f_hbm.at[pl.ds(mg, 1)],
                                 fbuf.at[0, pl.ds(w * L + t, 1)],
                                 gsem.at[0])

        for w in range(GCAP // L):
            fire0(w)

    lax.fori_loop(0, RPW, row, 0)

    for sl in range(2):
        pltpu.make_async_copy(slab.at[sl, :, pl.ds(0, X)],
                              out_hbm.at[0, :, 0, :], osem.at[sl]).wait()


_sc_call = functools.partial(
    pl.kernel,
    out_type=jax.ShapeDtypeStruct((B, C, Z, X), jnp.float32),
    mesh=plsc.VectorSubcoreMesh(core_axis_name="c", subcore_axis_name="s"),
    compiler_params=pltpu.CompilerParams(
        needs_layout_passes=False, use_tc_tiling_on_sc=False),
    scratch_types=[
        pltpu.VMEM((2, CH), jnp.int32),
        pltpu.VMEM((2, CH), jnp.int32),
        pltpu.VMEM((SCAP,), jnp.int32),
        pltpu.VMEM((TMPC,), jnp.int32),
        pltpu.VMEM((80,), jnp.int32),
        pltpu.VMEM((96,), jnp.int32),
        pltpu.VMEM((80,), jnp.int32),
        pltpu.VMEM((2, GCAP, C), jnp.float32),
        pltpu.VMEM((2, C, XP), jnp.float32),
        pltpu.VMEM((2, X + L), jnp.int32),
        pltpu.SemaphoreType.DMA((2,)),
        pltpu.SemaphoreType.DMA((2,)),
        pltpu.SemaphoreType.DMA((2,)),
    ],
)(_body)


def kernel(voxel_features, voxel_coords):
    z = voxel_coords[:, :, 0].reshape(-1)
    x = voxel_coords[:, :, 2].reshape(-1)
    f = voxel_features.reshape(B * M, C)
    return _sc_call(z, x, f)
